# baseline (speedup 1.0000x reference)
import functools

import jax
import jax.numpy as jnp
from jax import lax
from jax.experimental import pallas as pl
from jax.experimental.pallas import tpu as pltpu
from jax.experimental.pallas import tpu_sc as plsc

LN_EPS = 1e-7
NC = 2
NS = 16
NW = NC * NS
CH = 64


def _sc_gather_fn(B, S, V, H, L16):
    mesh = plsc.VectorSubcoreMesh(
        core_axis_name="c", subcore_axis_name="s", num_cores=NC, num_subcores=NS)
    ex_per_w = B // NW
    nch = S // CH

    @functools.partial(
        pl.kernel,
        out_type=(
            jax.ShapeDtypeStruct((B * S, H), jnp.float32),
            jax.ShapeDtypeStruct((B, H), jnp.float32),
            jax.ShapeDtypeStruct((B, H), jnp.float32),
            jax.ShapeDtypeStruct((B, H), jnp.float32),
        ),
        mesh=mesh,
        scratch_types=[
            pltpu.VMEM((CH,), jnp.int32),
            pltpu.VMEM((CH, H), jnp.float32),
            pltpu.VMEM((1, H), jnp.float32),
            pltpu.VMEM((16,), jnp.int32),
            pltpu.VMEM((16, H), jnp.float32),
            pltpu.SemaphoreType.DMA,
        ],
    )
    def k(ids_hbm, antidx_hbm, annidx_hbm, wtab, anttab, anntab,
          g_out, wsums, antrows, annsums,
          idx_v, rows_v, acc_v, idx16_v, rows16_v, sem):
        wid = lax.axis_index("s") * NC + lax.axis_index("c")
        nj = H // 16

        def zero_acc():
            zf32 = jnp.zeros((16,), jnp.float32)
            for j in range(nj):
                acc_v[0, pl.ds(j * 16, 16)] = zf32

        def accum_rows(src_ref, nrows):
            def row_body(r, carry):
                for j in range(nj):
                    plsc.addupdate(acc_v.at[0, pl.ds(j * 16, 16)],
                                   src_ref[r, pl.ds(j * 16, 16)])
                return carry
            lax.fori_loop(0, nrows, row_body, 0)

        for e in range(ex_per_w):
            b = wid * ex_per_w + e
            zero_acc()

            def chunk_body(c, carry, b=b):
                off = b * S + c * CH
                pltpu.sync_copy(ids_hbm.at[pl.ds(off, CH)], idx_v)
                pltpu.async_copy(wtab.at[idx_v], rows_v, sem).wait()
                pltpu.sync_copy(rows_v, g_out.at[pl.ds(off, CH)])
                accum_rows(rows_v, CH)
                return carry

            lax.fori_loop(0, nch, chunk_body, 0)
            pltpu.sync_copy(acc_v, wsums.at[pl.ds(b, 1)])

            pltpu.sync_copy(antidx_hbm.at[pl.ds(b * 16, 16)], idx16_v)
            pltpu.async_copy(anttab.at[idx16_v], rows16_v, sem).wait()
            pltpu.sync_copy(rows16_v.at[pl.ds(0, 1)], antrows.at[pl.ds(b, 1)])

            zero_acc()
            pltpu.sync_copy(annidx_hbm.at[pl.ds(b * 16, 16)], idx16_v)
            pltpu.async_copy(anntab.at[idx16_v], rows16_v, sem).wait()
            accum_rows(rows16_v, 16)
            pltpu.sync_copy(acc_v, annsums.at[pl.ds(b, 1)])

    return k


def _addvec_body(wsums_ref, antrows_ref, annsums_ref, pos_ref,
                 sentw_ref, antw_ref, annw_ref, out_ref, *, S, L):
    pos_mean = jnp.mean(pos_ref[...], axis=0, keepdims=True)
    sent_mean = wsums_ref[...] * (1.0 / S) + pos_mean
    ann_mean = annsums_ref[...] * (1.0 / L)
    dn = (((1,), (1,)), ((), ()))
    a_s = lax.dot_general(sent_mean, sentw_ref[...], dn,
                          preferred_element_type=jnp.float32)
    a_a = lax.dot_general(antrows_ref[...], antw_ref[...], dn,
                          preferred_element_type=jnp.float32)
    a_n = lax.dot_general(ann_mean, annw_ref[...], dn,
                          preferred_element_type=jnp.float32)
    alpha = jnp.sum(a_s * a_a, axis=1, keepdims=True)
    beta = jnp.sum(a_s * a_n, axis=1, keepdims=True)
    out_ref[...] = alpha * antrows_ref[...] + beta * ann_mean


def _ln_body(g_ref, pos_ref, addvec_ref, gamma_ref, beta_ref, out_ref):
    x = g_ref[...] + pos_ref[...] + addvec_ref[0]
    mu = jnp.mean(x, axis=1, keepdims=True)
    xc = x - mu
    var = jnp.mean(xc * xc, axis=1, keepdims=True)
    out_ref[...] = xc * lax.rsqrt(var + LN_EPS) * gamma_ref[...] + beta_ref[...]


def kernel(input_ids, annotator_ids, annotations, word_emb, pos_emb, sent_W,
           annotator_W, annotation_W, annotator_table, annotation_table,
           ln_gamma, ln_beta):
    B, S = input_ids.shape
    V, H = word_emb.shape
    L = annotations.shape[1]

    ids_flat = input_ids.reshape(-1).astype(jnp.int32)
    ant_idx16 = jnp.concatenate(
        [annotator_ids[:, None].astype(jnp.int32),
         jnp.zeros((B, 15), jnp.int32)], axis=1).reshape(-1)
    ann_idx16 = jnp.concatenate(
        [annotations.astype(jnp.int32),
         jnp.zeros((B, 16 - L), jnp.int32)], axis=1).reshape(-1)

    sc = _sc_gather_fn(B, S, V, H, 16)
    g, wsums, antrows, annsums = sc(
        ids_flat, ant_idx16, ann_idx16, word_emb, annotator_table,
        annotation_table)

    addvec = pl.pallas_call(
        functools.partial(_addvec_body, S=S, L=L),
        out_shape=jax.ShapeDtypeStruct((B, H), jnp.float32),
    )(wsums, antrows, annsums, pos_emb, sent_W, annotator_W, annotation_W)

    gamma2 = ln_gamma.reshape(1, H)
    beta2 = ln_beta.reshape(1, H)
    out = pl.pallas_call(
        _ln_body,
        grid=(B,),
        in_specs=[
            pl.BlockSpec((S, H), lambda i: (i, 0)),
            pl.BlockSpec((S, H), lambda i: (0, 0)),
            pl.BlockSpec((1, 1, H), lambda i: (i, 0, 0)),
            pl.BlockSpec((1, H), lambda i: (0, 0)),
            pl.BlockSpec((1, H), lambda i: (0, 0)),
        ],
        out_specs=pl.BlockSpec((S, H), lambda i: (i, 0)),
        out_shape=jax.ShapeDtypeStruct((B * S, H), jnp.float32),
        compiler_params=pltpu.CompilerParams(
            dimension_semantics=("arbitrary",)),
    )(g, pos_emb, addvec.reshape(B, 1, H), gamma2, beta2)

    return out.reshape(B, S, H)


# speedup vs baseline: 1.3337x; 1.3337x over previous
import functools

import jax
import jax.numpy as jnp
from jax import lax
from jax.experimental import pallas as pl
from jax.experimental.pallas import tpu as pltpu
from jax.experimental.pallas import tpu_sc as plsc

LN_EPS = 1e-7
NC = 2
NS = 16
NW = NC * NS
CH = 64


def _sc_gather_fn(B, S, H):
    mesh = plsc.VectorSubcoreMesh(
        core_axis_name="c", subcore_axis_name="s", num_cores=NC, num_subcores=NS)
    ex_per_w = B // NW
    nch = S // CH

    @functools.partial(
        pl.kernel,
        out_type=(
            jax.ShapeDtypeStruct((B * S, H), jnp.float32),
            jax.ShapeDtypeStruct((B, H), jnp.float32),
            jax.ShapeDtypeStruct((B, H), jnp.float32),
        ),
        mesh=mesh,
        scratch_types=[
            pltpu.VMEM((S,), jnp.int32),
            pltpu.VMEM((CH, H), jnp.float32),
            pltpu.VMEM((1, H), jnp.float32),
            pltpu.VMEM((16,), jnp.int32),
            pltpu.VMEM((16, H), jnp.float32),
            pltpu.SemaphoreType.DMA,
        ],
    )
    def k(ids_hbm, antidx_hbm, annidx_hbm, wtab, anttab, anntab,
          g_out, antrows, annsums,
          idx_ex, rows_v, acc_v, idx16_v, rows16_v, sem):
        wid = lax.axis_index("s") * NC + lax.axis_index("c")
        nj = H // 16

        for e in range(ex_per_w):
            b = wid * ex_per_w + e
            pltpu.sync_copy(ids_hbm.at[pl.ds(b * S, S)], idx_ex)

            def chunk_body(c, carry, b=b):
                off = b * S + c * CH
                pltpu.async_copy(
                    wtab.at[idx_ex.at[pl.ds(c * CH, CH)]], rows_v, sem).wait()
                pltpu.sync_copy(rows_v, g_out.at[pl.ds(off, CH)])
                return carry

            lax.fori_loop(0, nch, chunk_body, 0)

            pltpu.sync_copy(antidx_hbm.at[pl.ds(b * 16, 16)], idx16_v)
            pltpu.async_copy(anttab.at[idx16_v], rows16_v, sem).wait()
            pltpu.sync_copy(rows16_v.at[pl.ds(0, 1)], antrows.at[pl.ds(b, 1)])

            zf32 = jnp.zeros((16,), jnp.float32)
            for j in range(nj):
                acc_v[0, pl.ds(j * 16, 16)] = zf32
            pltpu.sync_copy(annidx_hbm.at[pl.ds(b * 16, 16)], idx16_v)
            pltpu.async_copy(anntab.at[idx16_v], rows16_v, sem).wait()

            def row_body(r, carry):
                for j in range(nj):
                    plsc.addupdate(acc_v.at[0, pl.ds(j * 16, 16)],
                                   rows16_v[r, pl.ds(j * 16, 16)])
                return carry

            lax.fori_loop(0, 16, row_body, 0)
            pltpu.sync_copy(acc_v, annsums.at[pl.ds(b, 1)])

    return k


def _fused_body(g_ref, pos_ref, antrows_ref, annsums_ref,
                sentw_ref, antw_ref, annw_ref, gamma_ref, lnbeta_ref,
                out_ref, wsums_ref, addvec_ref, *, S, L, B):
    p = pl.program_id(0)
    b = pl.program_id(1)

    @pl.when(p == 0)
    def _sums():
        wsums_ref[pl.ds(b, 1)] = jnp.sum(g_ref[...], axis=0, keepdims=True)

    @pl.when(jnp.logical_and(p == 1, b == 0))
    def _gates():
        pos_mean = jnp.mean(pos_ref[...], axis=0, keepdims=True)
        sent_mean = wsums_ref[...] * (1.0 / S) + pos_mean
        ann_mean = annsums_ref[...] * (1.0 / L)
        dn = (((1,), (1,)), ((), ()))
        a_s = lax.dot_general(sent_mean, sentw_ref[...], dn,
                              preferred_element_type=jnp.float32)
        a_a = lax.dot_general(antrows_ref[...], antw_ref[...], dn,
                              preferred_element_type=jnp.float32)
        a_n = lax.dot_general(ann_mean, annw_ref[...], dn,
                              preferred_element_type=jnp.float32)
        alpha = jnp.sum(a_s * a_a, axis=1, keepdims=True)
        beta = jnp.sum(a_s * a_n, axis=1, keepdims=True)
        addvec_ref[...] = alpha * antrows_ref[...] + beta * ann_mean

    @pl.when(p == 1)
    def _ln():
        x = g_ref[...] + pos_ref[...] + addvec_ref[pl.ds(b, 1)]
        mu = jnp.mean(x, axis=1, keepdims=True)
        xc = x - mu
        var = jnp.mean(xc * xc, axis=1, keepdims=True)
        out_ref[...] = (xc * lax.rsqrt(var + LN_EPS) * gamma_ref[...]
                        + lnbeta_ref[...])


def kernel(input_ids, annotator_ids, annotations, word_emb, pos_emb, sent_W,
           annotator_W, annotation_W, annotator_table, annotation_table,
           ln_gamma, ln_beta):
    B, S = input_ids.shape
    V, H = word_emb.shape
    L = annotations.shape[1]

    ids_flat = input_ids.reshape(-1).astype(jnp.int32)
    ant_idx16 = jnp.concatenate(
        [annotator_ids[:, None].astype(jnp.int32),
         jnp.zeros((B, 15), jnp.int32)], axis=1).reshape(-1)
    ann_idx16 = jnp.concatenate(
        [annotations.astype(jnp.int32),
         jnp.zeros((B, 16 - L), jnp.int32)], axis=1).reshape(-1)

    sc = _sc_gather_fn(B, S, H)
    g, antrows, annsums = sc(
        ids_flat, ant_idx16, ann_idx16, word_emb, annotator_table,
        annotation_table)

    c0 = lambda p, b: (0, 0)
    out = pl.pallas_call(
        functools.partial(_fused_body, S=S, L=L, B=B),
        grid=(2, B),
        in_specs=[
            pl.BlockSpec((S, H), lambda p, b: (b, 0)),
            pl.BlockSpec((S, H), c0),
            pl.BlockSpec((B, H), c0),
            pl.BlockSpec((B, H), c0),
            pl.BlockSpec((H, H), c0),
            pl.BlockSpec((H, H), c0),
            pl.BlockSpec((H, H), c0),
            pl.BlockSpec((1, H), c0),
            pl.BlockSpec((1, H), c0),
        ],
        out_specs=pl.BlockSpec(
            (S, H), lambda p, b: (jnp.where(p == 1, b, 0), 0)),
        out_shape=jax.ShapeDtypeStruct((B * S, H), jnp.float32),
        scratch_shapes=[
            pltpu.VMEM((B, H), jnp.float32),
            pltpu.VMEM((B, H), jnp.float32),
        ],
        compiler_params=pltpu.CompilerParams(
            dimension_semantics=("arbitrary", "arbitrary")),
    )(g, pos_emb, antrows, annsums, sent_W, annotator_W, annotation_W,
      ln_gamma.reshape(1, H), ln_beta.reshape(1, H))

    return out.reshape(B, S, H)


# speedup vs baseline: 1.6815x; 1.2608x over previous
import functools

import jax
import jax.numpy as jnp
from jax import lax
from jax.experimental import pallas as pl
from jax.experimental.pallas import tpu as pltpu
from jax.experimental.pallas import tpu_sc as plsc

LN_EPS = 1e-7
NC = 2
NS = 16
NW = NC * NS
CH = 64


def _sc_gather_fn(B, S, H):
    mesh = plsc.VectorSubcoreMesh(
        core_axis_name="c", subcore_axis_name="s", num_cores=NC, num_subcores=NS)
    ex_per_w = B // NW
    nch = S // CH

    @functools.partial(
        pl.kernel,
        out_type=(
            jax.ShapeDtypeStruct((B * S, H), jnp.float32),
            jax.ShapeDtypeStruct((B, H), jnp.float32),
            jax.ShapeDtypeStruct((B, H), jnp.float32),
        ),
        mesh=mesh,
        scratch_types=[
            pltpu.VMEM((S,), jnp.int32),
            pltpu.VMEM((CH, H), jnp.float32),
            pltpu.VMEM((CH, H), jnp.float32),
            pltpu.VMEM((1, H), jnp.float32),
            pltpu.VMEM((16,), jnp.int32),
            pltpu.VMEM((16, H), jnp.float32),
            pltpu.SemaphoreType.DMA,
            pltpu.SemaphoreType.DMA,
        ],
    )
    def k(ids_hbm, antidx_hbm, annidx_hbm, wtab, anttab, anntab,
          g_out, antrows, annsums,
          idx_ex, rows0_v, rows1_v, acc_v, idx16_v, rows16_v, gsem0, gsem1):
        wid = lax.axis_index("s") * NC + lax.axis_index("c")
        nj = H // 16

        for e in range(ex_per_w):
            b = wid * ex_per_w + e
            pltpu.sync_copy(ids_hbm.at[pl.ds(b * S, S)], idx_ex)

            def gather(c, buf, sem):
                pltpu.async_copy(
                    wtab.at[idx_ex.at[pl.ds(c * CH, CH)]], buf, sem)

            def gwait(buf, sem):
                pltpu.make_async_copy(
                    wtab.at[idx_ex.at[pl.ds(0, CH)]], buf, sem).wait()

            gather(0, rows0_v, gsem0)
            gather(1, rows1_v, gsem1)

            def pair_body(i, carry, b=b):
                c0 = 2 * i
                gwait(rows0_v, gsem0)
                pltpu.sync_copy(rows0_v, g_out.at[pl.ds(b * S + c0 * CH, CH)])
                gather(c0 + 2, rows0_v, gsem0)
                gwait(rows1_v, gsem1)
                pltpu.sync_copy(rows1_v,
                                g_out.at[pl.ds(b * S + (c0 + 1) * CH, CH)])
                gather(c0 + 3, rows1_v, gsem1)
                return carry

            lax.fori_loop(0, nch // 2 - 1, pair_body, 0)

            cl = nch - 2
            gwait(rows0_v, gsem0)
            pltpu.sync_copy(rows0_v, g_out.at[pl.ds(b * S + cl * CH, CH)])
            gwait(rows1_v, gsem1)
            pltpu.sync_copy(rows1_v, g_out.at[pl.ds(b * S + (cl + 1) * CH, CH)])

            pltpu.sync_copy(antidx_hbm.at[pl.ds(b * 16, 16)], idx16_v)
            pltpu.async_copy(anttab.at[idx16_v], rows16_v, gsem0).wait()
            pltpu.sync_copy(rows16_v.at[pl.ds(0, 1)], antrows.at[pl.ds(b, 1)])

            zf32 = jnp.zeros((16,), jnp.float32)
            for j in range(nj):
                acc_v[0, pl.ds(j * 16, 16)] = zf32
            pltpu.sync_copy(annidx_hbm.at[pl.ds(b * 16, 16)], idx16_v)
            pltpu.async_copy(anntab.at[idx16_v], rows16_v, gsem0).wait()

            def row_body(r, carry):
                for j in range(nj):
                    plsc.addupdate(acc_v.at[0, pl.ds(j * 16, 16)],
                                   rows16_v[r, pl.ds(j * 16, 16)])
                return carry

            lax.fori_loop(0, 16, row_body, 0)
            pltpu.sync_copy(acc_v, annsums.at[pl.ds(b, 1)])

    return k


def _fused_body(g_ref, pos_ref, antrows_ref, annsums_ref,
                sentw_ref, antw_ref, annw_ref, gamma_ref, lnbeta_ref,
                out_ref, u_ref, v_ref, pm_ref, am_ref, *, S, L):
    b = pl.program_id(0)

    @pl.when(b == 0)
    def _precompute():
        pm_ref[...] = jnp.mean(pos_ref[...], axis=0, keepdims=True)
        am_ref[...] = annsums_ref[...] * (1.0 / L)
        dt = (((1,), (1,)), ((), ()))
        dp = (((1,), (0,)), ((), ()))
        u_ref[...] = lax.dot_general(
            lax.dot_general(antrows_ref[...], antw_ref[...], dt,
                            preferred_element_type=jnp.float32),
            sentw_ref[...], dp, preferred_element_type=jnp.float32)
        v_ref[...] = lax.dot_general(
            lax.dot_general(am_ref[...], annw_ref[...], dt,
                            preferred_element_type=jnp.float32),
            sentw_ref[...], dp, preferred_element_type=jnp.float32)

    g = g_ref[...]
    m = jnp.sum(g, axis=0, keepdims=True) * (1.0 / S) + pm_ref[...]
    alpha = jnp.sum(m * u_ref[pl.ds(b, 1)], axis=1, keepdims=True)
    beta = jnp.sum(m * v_ref[pl.ds(b, 1)], axis=1, keepdims=True)
    addvec = alpha * antrows_ref[pl.ds(b, 1)] + beta * am_ref[pl.ds(b, 1)]
    x = g + pos_ref[...] + addvec
    mu = jnp.mean(x, axis=1, keepdims=True)
    xc = x - mu
    var = jnp.mean(xc * xc, axis=1, keepdims=True)
    out_ref[...] = xc * lax.rsqrt(var + LN_EPS) * gamma_ref[...] + lnbeta_ref[...]


def kernel(input_ids, annotator_ids, annotations, word_emb, pos_emb, sent_W,
           annotator_W, annotation_W, annotator_table, annotation_table,
           ln_gamma, ln_beta):
    B, S = input_ids.shape
    V, H = word_emb.shape
    L = annotations.shape[1]

    ids_flat = input_ids.reshape(-1).astype(jnp.int32)
    ant_idx16 = jnp.concatenate(
        [annotator_ids[:, None].astype(jnp.int32),
         jnp.zeros((B, 15), jnp.int32)], axis=1).reshape(-1)
    ann_idx16 = jnp.concatenate(
        [annotations.astype(jnp.int32),
         jnp.zeros((B, 16 - L), jnp.int32)], axis=1).reshape(-1)

    sc = _sc_gather_fn(B, S, H)
    g, antrows, annsums = sc(
        ids_flat, ant_idx16, ann_idx16, word_emb, annotator_table,
        annotation_table)

    c0 = lambda b: (0, 0)
    out = pl.pallas_call(
        functools.partial(_fused_body, S=S, L=L),
        grid=(B,),
        in_specs=[
            pl.BlockSpec((S, H), lambda b: (b, 0)),
            pl.BlockSpec((S, H), c0),
            pl.BlockSpec((B, H), c0),
            pl.BlockSpec((B, H), c0),
            pl.BlockSpec((H, H), c0),
            pl.BlockSpec((H, H), c0),
            pl.BlockSpec((H, H), c0),
            pl.BlockSpec((1, H), c0),
            pl.BlockSpec((1, H), c0),
        ],
        out_specs=pl.BlockSpec((S, H), lambda b: (b, 0)),
        out_shape=jax.ShapeDtypeStruct((B * S, H), jnp.float32),
        scratch_shapes=[
            pltpu.VMEM((B, H), jnp.float32),
            pltpu.VMEM((B, H), jnp.float32),
            pltpu.VMEM((1, H), jnp.float32),
            pltpu.VMEM((B, H), jnp.float32),
        ],
        compiler_params=pltpu.CompilerParams(
            dimension_semantics=("arbitrary",)),
    )(g, pos_emb, antrows, annsums, sent_W, annotator_W, annotation_W,
      ln_gamma.reshape(1, H), ln_beta.reshape(1, H))

    return out.reshape(B, S, H)
